# baseline (speedup 1.0000x reference)
import jax
import jax.numpy as jnp
from jax.experimental import pallas as pl


def _mean_kernel(x_ref, o_ref):
    o_ref[...] = jnp.mean(x_ref[...])[None, None]


def kernel(vertex_features, edges, weights, W1, b1, W2, b2, W3, b3, W4, b4, W5, b5):
    n = vertex_features.shape[0]
    src, dst = edges[0], edges[1]
    x = vertex_features
    loop = jnp.arange(n, dtype=src.dtype)
    s = jnp.concatenate([src, loop])
    d = jnp.concatenate([dst, loop])
    wf = jnp.concatenate([weights, jnp.ones((n,), dtype=weights.dtype)])
    deg = jax.ops.segment_sum(wf, d, num_segments=n)
    dinv = deg ** -0.5
    norm = dinv[s] * wf * dinv[d]
    for W, b in ((W1, b1), (W2, b2), (W3, b3), (W4, b4), (W5, b5)):
        h = x @ W.T
        msg = h[s] * norm[:, None]
        out = jax.ops.segment_sum(msg, d, num_segments=n)
        x = jax.nn.sigmoid(out + b)
    xp = jnp.pad(x, ((0, 100352 - n), (0, 0)))
    return pl.pallas_call(
        _mean_kernel,
        out_shape=jax.ShapeDtypeStruct((1, 1), jnp.float32),
    )(xp) * (100352.0 / n)


# speedup vs baseline: 4.2006x; 4.2006x over previous
import functools

import jax
import jax.numpy as jnp
from jax import lax
from jax.experimental import pallas as pl
from jax.experimental.pallas import tpu as pltpu
from jax.experimental.pallas import tpu_sc as plsc

N = 100000
E = 1600000
HID = 64
RP = 13312
EPAD = RP * 128 - (E + N)
NC, NS = 2, 16
ROWS_SC = RP // NC
ROWS_TILE = ROWS_SC // NS
N2 = 100096
NTILE = N2 // NS
ZROWS = NTILE // 8

_MESH = plsc.VectorSubcoreMesh(
    core_axis_name="c", subcore_axis_name="s", num_cores=NC, num_subcores=NS)

_f32 = jnp.float32


def _prop_body(nchunks, *refs):
    (src_hbm, dst_hbm, nrm_hbm), tbls = refs[:3], refs[3:3 + nchunks]
    out_hbm = refs[3 + nchunks]
    acc, zbuf, srcv, dstv, nrmv, rows, sem = refs[4 + nchunks:]
    cid = lax.axis_index("c")
    sid = lax.axis_index("s")

    def zb(i, _):
        zbuf[i, :] = jnp.zeros((16,), _f32)
        return 0

    lax.fori_loop(0, ZROWS, zb, 0)

    row0 = cid * ROWS_SC + sid * ROWS_TILE
    for c in range(nchunks):
        tbl = tbls[c]
        for z in range(8):
            pltpu.sync_copy(zbuf, acc.at[pl.ds(sid * NTILE + z * ZROWS, ZROWS)])
        plsc.subcore_barrier()

        def body(t, _):
            r = row0 + t
            pltpu.sync_copy(src_hbm.at[pl.ds(r * 128, 128)], srcv)
            pltpu.sync_copy(dst_hbm.at[pl.ds(r * 128, 128)], dstv)
            pltpu.sync_copy(nrm_hbm.at[pl.ds(r * 128, 128)], nrmv)
            pltpu.async_copy(tbl.at[srcv], rows, sem).wait()

            for k in range(8):
                nv = nrmv[pl.ds(k * 16, 16)]
                for j in range(16):
                    rows[k * 16 + j, :] = rows[k * 16 + j, :] * nv[j]
            pltpu.sync_copy(rows, acc.at[dstv], add=True)
            return 0

        lax.fori_loop(0, ROWS_TILE, body, 0)
        plsc.subcore_barrier()
        pltpu.sync_copy(acc.at[pl.ds(sid * NTILE, NTILE)],
                        out_hbm.at[cid, c, pl.ds(sid * NTILE, NTILE)])


def _make_prop(nchunks):
    return pl.kernel(
        functools.partial(_prop_body, nchunks),
        out_type=jax.ShapeDtypeStruct((NC, nchunks, N2, 16), _f32),
        mesh=_MESH,
        compiler_params=pltpu.CompilerParams(use_tc_tiling_on_sc=False),
        scratch_types=[
            pltpu.VMEM_SHARED((N2, 16), _f32),
            pltpu.VMEM((ZROWS, 16), _f32),
            pltpu.VMEM((128,), jnp.int32),
            pltpu.VMEM((128,), jnp.int32),
            pltpu.VMEM((128,), _f32),
            pltpu.VMEM((128, 16), _f32),
            pltpu.SemaphoreType.DMA,
        ],
    )


_sc_prop1 = _make_prop(1)
_sc_prop4 = _make_prop(4)


_BR = 1000


def _dinv_body(degm_ref, x0_ref, dinv_ref, g1_ref):
    deg = degm_ref[0, 0] + degm_ref[1, 0] + 1.0
    dinv = lax.rsqrt(deg)
    dinv_ref[...] = dinv
    g1_ref[...] = dinv * x0_ref[...]


_tc_dinv = pl.pallas_call(
    _dinv_body,
    grid=(N // _BR,),
    in_specs=[
        pl.BlockSpec((NC, 1, _BR, 16), lambda i: (0, 0, i, 0)),
        pl.BlockSpec((_BR, 16), lambda i: (i, 0)),
    ],
    out_specs=[
        pl.BlockSpec((_BR, 16), lambda i: (i, 0)),
        pl.BlockSpec((_BR, 16), lambda i: (i, 0)),
    ],
    out_shape=[
        jax.ShapeDtypeStruct((N, 16), _f32),
        jax.ShapeDtypeStruct((N, 16), _f32),
    ],
)


def _layer1_body(r_ref, dinv_ref, w1_ref, b1_ref, w2_ref, *o_refs):
    dinv = dinv_ref[...]
    t = dinv * (r_ref[0, 0] + r_ref[1, 0])
    x1 = jax.nn.sigmoid(
        jnp.dot(t, w1_ref[...].T, preferred_element_type=_f32) + b1_ref[...])
    h2 = jnp.dot(x1, w2_ref[...].T, preferred_element_type=_f32)
    for c in range(4):
        o_refs[c][...] = dinv * h2[:, c * 16:(c + 1) * 16]


_tc_layer1 = pl.pallas_call(
    _layer1_body,
    grid=(N // _BR,),
    in_specs=[
        pl.BlockSpec((NC, 1, _BR, 16), lambda i: (0, 0, i, 0)),
        pl.BlockSpec((_BR, 16), lambda i: (i, 0)),
        pl.BlockSpec((HID, 16), lambda i: (0, 0)),
        pl.BlockSpec((1, HID), lambda i: (0, 0)),
        pl.BlockSpec((HID, HID), lambda i: (0, 0)),
    ],
    out_specs=[pl.BlockSpec((_BR, 16), lambda i: (i, 0)) for _ in range(4)],
    out_shape=[jax.ShapeDtypeStruct((N, 16), _f32) for _ in range(4)],
)


def _layer_mid_body(r_ref, dinv_ref, b_ref, w_ref, *o_refs):
    dinv = dinv_ref[...]
    t = jnp.concatenate(
        [dinv * (r_ref[0, c] + r_ref[1, c]) for c in range(4)], axis=-1)
    x = jax.nn.sigmoid(t + b_ref[...])
    h = jnp.dot(x, w_ref[...].T, preferred_element_type=_f32)
    for c in range(4):
        o_refs[c][...] = dinv * h[:, c * 16:(c + 1) * 16]


_tc_layer_mid = pl.pallas_call(
    _layer_mid_body,
    grid=(N // _BR,),
    in_specs=[
        pl.BlockSpec((NC, 4, _BR, 16), lambda i: (0, 0, i, 0)),
        pl.BlockSpec((_BR, 16), lambda i: (i, 0)),
        pl.BlockSpec((1, HID), lambda i: (0, 0)),
        pl.BlockSpec((HID, HID), lambda i: (0, 0)),
    ],
    out_specs=[pl.BlockSpec((_BR, 16), lambda i: (i, 0)) for _ in range(4)],
    out_shape=[jax.ShapeDtypeStruct((N, 16), _f32) for _ in range(4)],
)


def _layer4_body(r_ref, dinv_ref, b_ref, w5_ref, o_ref):
    dinv = dinv_ref[...]
    t = jnp.concatenate(
        [dinv * (r_ref[0, c] + r_ref[1, c]) for c in range(4)], axis=-1)
    x = jax.nn.sigmoid(t + b_ref[...])
    h5 = jnp.dot(x, w5_ref[...].T, preferred_element_type=_f32)
    o_ref[...] = jnp.concatenate(
        [dinv[:, 0:1] * h5, jnp.zeros((_BR, 15), _f32)], axis=-1)


_tc_layer4 = pl.pallas_call(
    _layer4_body,
    grid=(N // _BR,),
    in_specs=[
        pl.BlockSpec((NC, 4, _BR, 16), lambda i: (0, 0, i, 0)),
        pl.BlockSpec((_BR, 16), lambda i: (i, 0)),
        pl.BlockSpec((1, HID), lambda i: (0, 0)),
        pl.BlockSpec((1, HID), lambda i: (0, 0)),
    ],
    out_specs=pl.BlockSpec((_BR, 16), lambda i: (i, 0)),
    out_shape=jax.ShapeDtypeStruct((N, 16), _f32),
)


def _final_body(r_ref, dinv_ref, b_ref, o_ref):
    i = pl.program_id(0)
    a = dinv_ref[:, 0:1] * (r_ref[0, 0, :, 0:1] + r_ref[1, 0, :, 0:1])
    v = jax.nn.sigmoid(a + b_ref[0, 0])

    @pl.when(i == 0)
    def _():
        o_ref[...] = jnp.zeros_like(o_ref)

    o_ref[...] += (jnp.sum(v) / N)[None, None]


_tc_final = pl.pallas_call(
    _final_body,
    grid=(N // _BR,),
    in_specs=[
        pl.BlockSpec((NC, 1, _BR, 16), lambda i: (0, 0, i, 0)),
        pl.BlockSpec((_BR, 16), lambda i: (i, 0)),
        pl.BlockSpec((1, 1), lambda i: (0, 0)),
    ],
    out_specs=pl.BlockSpec((1, 1), lambda i: (0, 0)),
    out_shape=jax.ShapeDtypeStruct((1, 1), _f32),
)


def kernel(vertex_features, edges, weights, W1, b1, W2, b2, W3, b3, W4, b4,
           W5, b5):
    src = edges[0].astype(jnp.int32)
    dst = edges[1].astype(jnp.int32)

    ar = jnp.arange(N, dtype=jnp.int32)
    zi = jnp.zeros((EPAD,), jnp.int32)
    srcP = jnp.concatenate([src, ar, zi])
    dstP = jnp.concatenate([dst, ar, zi])
    nrmD = jnp.concatenate([weights, jnp.zeros((N + EPAD,), _f32)])
    nrmP = jnp.concatenate(
        [weights, jnp.ones((N,), _f32), jnp.zeros((EPAD,), _f32)])

    ones_tbl = jnp.ones((N, 16), _f32)
    x0 = jnp.pad(vertex_features, ((0, 0), (0, 16 - 6)))
    W1p = jnp.pad(W1, ((0, 0), (0, 16 - 6)))

    degm = _sc_prop1(srcP, dstP, nrmD, ones_tbl)
    dinv, g1 = _tc_dinv(degm, x0)
    r1 = _sc_prop1(srcP, dstP, nrmP, g1)
    g2 = _tc_layer1(r1, dinv, W1p, b1.reshape(1, HID), W2)
    r2 = _sc_prop4(srcP, dstP, nrmP, *g2)
    g3 = _tc_layer_mid(r2, dinv, b2.reshape(1, HID), W3)
    r3 = _sc_prop4(srcP, dstP, nrmP, *g3)
    g4 = _tc_layer_mid(r3, dinv, b3.reshape(1, HID), W4)
    r4 = _sc_prop4(srcP, dstP, nrmP, *g4)
    g5 = _tc_layer4(r4, dinv, b4.reshape(1, HID), W5)
    r5 = _sc_prop1(srcP, dstP, nrmP, g5)
    return _tc_final(r5, dinv, b5.reshape(1, 1))


# speedup vs baseline: 10.9183x; 2.5992x over previous
import functools

import jax
import jax.numpy as jnp
from jax import lax
from jax.experimental import pallas as pl
from jax.experimental.pallas import tpu as pltpu
from jax.experimental.pallas import tpu_sc as plsc

N = 100000
E = 1600000
HID = 64
RP = 13312
EPAD = RP * 128 - (E + N)
NC, NS = 2, 16
ROWS_SC = RP // NC
ROWS_TILE = ROWS_SC // NS
N2 = 100096
NTILE = N2 // NS
GROUP = 4
ZROWS = NTILE // 8

_MESH = plsc.VectorSubcoreMesh(
    core_axis_name="c", subcore_axis_name="s", num_cores=NC, num_subcores=NS)

_f32 = jnp.float32


def _prop_body(nchunks, *refs):
    (src_hbm, dst_hbm, nrm_hbm), tbls = refs[:3], refs[3:3 + nchunks]
    out_hbm = refs[3 + nchunks]
    rest = refs[4 + nchunks:]
    acc, zbuf = rest[0], rest[1]
    srcv = rest[2:2 + GROUP]
    dstv = rest[2 + GROUP:2 + 2 * GROUP]
    rows = rest[2 + 2 * GROUP:2 + 3 * GROUP]
    nrmv, semE, semG = rest[2 + 3 * GROUP:]
    cid = lax.axis_index("c")
    sid = lax.axis_index("s")

    def zb(i, _):
        zbuf[i, :] = jnp.zeros((16,), _f32)
        return 0

    lax.fori_loop(0, ZROWS, zb, 0)

    row0 = cid * ROWS_SC + sid * ROWS_TILE
    for c in range(nchunks):
        tbl = tbls[c]
        for z in range(8):
            pltpu.sync_copy(zbuf, acc.at[pl.ds(sid * NTILE + z * ZROWS, ZROWS)])
        plsc.subcore_barrier()

        def body(t, _):
            base = (row0 + t * GROUP) * 128
            eds = []
            for g in range(GROUP):
                eds.append(pltpu.async_copy(
                    src_hbm.at[pl.ds(base + g * 128, 128)], srcv[g], semE))
                eds.append(pltpu.async_copy(
                    dst_hbm.at[pl.ds(base + g * 128, 128)], dstv[g], semE))
            eds.append(pltpu.async_copy(
                nrm_hbm.at[pl.ds(base, GROUP * 128)], nrmv, semE))
            for e in eds:
                e.wait()
            cps = [pltpu.async_copy(tbl.at[srcv[g]], rows[g], semG)
                   for g in range(GROUP)]
            for g in range(GROUP):
                cps[g].wait()
                for k in range(8):
                    nv = nrmv[pl.ds((g * 8 + k) * 16, 16)]
                    for j in range(16):
                        rows[g][k * 16 + j, :] = rows[g][k * 16 + j, :] * nv[j]
                pltpu.sync_copy(rows[g], acc.at[dstv[g]], add=True)
            return 0

        lax.fori_loop(0, ROWS_TILE // GROUP, body, 0)
        plsc.subcore_barrier()
        pltpu.sync_copy(acc.at[pl.ds(sid * NTILE, NTILE)],
                        out_hbm.at[cid, c, pl.ds(sid * NTILE, NTILE)])


def _make_prop(nchunks):
    return pl.kernel(
        functools.partial(_prop_body, nchunks),
        out_type=jax.ShapeDtypeStruct((NC, nchunks, N2, 16), _f32),
        mesh=_MESH,
        compiler_params=pltpu.CompilerParams(use_tc_tiling_on_sc=False),
        scratch_types=(
            [pltpu.VMEM_SHARED((N2, 16), _f32),
             pltpu.VMEM((ZROWS, 16), _f32)]
            + [pltpu.VMEM((128,), jnp.int32) for _ in range(2 * GROUP)]
            + [pltpu.VMEM((128, 16), _f32) for _ in range(GROUP)]
            + [pltpu.VMEM((GROUP * 128,), _f32),
               pltpu.SemaphoreType.DMA,
               pltpu.SemaphoreType.DMA]
        ),
    )


_sc_prop1 = _make_prop(1)
_sc_prop4 = _make_prop(4)


_BR = 1000


def _dinv_body(degm_ref, x0_ref, dinv_ref, g1_ref):
    deg = degm_ref[0, 0] + degm_ref[1, 0] + 1.0
    dinv = lax.rsqrt(deg)
    dinv_ref[...] = dinv
    g1_ref[...] = dinv * x0_ref[...]


_tc_dinv = pl.pallas_call(
    _dinv_body,
    grid=(N // _BR,),
    in_specs=[
        pl.BlockSpec((NC, 1, _BR, 16), lambda i: (0, 0, i, 0)),
        pl.BlockSpec((_BR, 16), lambda i: (i, 0)),
    ],
    out_specs=[
        pl.BlockSpec((_BR, 16), lambda i: (i, 0)),
        pl.BlockSpec((_BR, 16), lambda i: (i, 0)),
    ],
    out_shape=[
        jax.ShapeDtypeStruct((N, 16), _f32),
        jax.ShapeDtypeStruct((N, 16), _f32),
    ],
)


def _layer1_body(r_ref, dinv_ref, w1_ref, b1_ref, w2_ref, *o_refs):
    dinv = dinv_ref[...]
    t = dinv * (r_ref[0, 0] + r_ref[1, 0])
    x1 = jax.nn.sigmoid(
        jnp.dot(t, w1_ref[...].T, preferred_element_type=_f32) + b1_ref[...])
    h2 = jnp.dot(x1, w2_ref[...].T, preferred_element_type=_f32)
    for c in range(4):
        o_refs[c][...] = dinv * h2[:, c * 16:(c + 1) * 16]


_tc_layer1 = pl.pallas_call(
    _layer1_body,
    grid=(N // _BR,),
    in_specs=[
        pl.BlockSpec((NC, 1, _BR, 16), lambda i: (0, 0, i, 0)),
        pl.BlockSpec((_BR, 16), lambda i: (i, 0)),
        pl.BlockSpec((HID, 16), lambda i: (0, 0)),
        pl.BlockSpec((1, HID), lambda i: (0, 0)),
        pl.BlockSpec((HID, HID), lambda i: (0, 0)),
    ],
    out_specs=[pl.BlockSpec((_BR, 16), lambda i: (i, 0)) for _ in range(4)],
    out_shape=[jax.ShapeDtypeStruct((N, 16), _f32) for _ in range(4)],
)


def _layer_mid_body(r_ref, dinv_ref, b_ref, w_ref, *o_refs):
    dinv = dinv_ref[...]
    t = jnp.concatenate(
        [dinv * (r_ref[0, c] + r_ref[1, c]) for c in range(4)], axis=-1)
    x = jax.nn.sigmoid(t + b_ref[...])
    h = jnp.dot(x, w_ref[...].T, preferred_element_type=_f32)
    for c in range(4):
        o_refs[c][...] = dinv * h[:, c * 16:(c + 1) * 16]


_tc_layer_mid = pl.pallas_call(
    _layer_mid_body,
    grid=(N // _BR,),
    in_specs=[
        pl.BlockSpec((NC, 4, _BR, 16), lambda i: (0, 0, i, 0)),
        pl.BlockSpec((_BR, 16), lambda i: (i, 0)),
        pl.BlockSpec((1, HID), lambda i: (0, 0)),
        pl.BlockSpec((HID, HID), lambda i: (0, 0)),
    ],
    out_specs=[pl.BlockSpec((_BR, 16), lambda i: (i, 0)) for _ in range(4)],
    out_shape=[jax.ShapeDtypeStruct((N, 16), _f32) for _ in range(4)],
)


def _layer4_body(r_ref, dinv_ref, b_ref, w5_ref, o_ref):
    dinv = dinv_ref[...]
    t = jnp.concatenate(
        [dinv * (r_ref[0, c] + r_ref[1, c]) for c in range(4)], axis=-1)
    x = jax.nn.sigmoid(t + b_ref[...])
    h5 = jnp.dot(x, w5_ref[...].T, preferred_element_type=_f32)
    o_ref[...] = jnp.concatenate(
        [dinv[:, 0:1] * h5, jnp.zeros((_BR, 15), _f32)], axis=-1)


_tc_layer4 = pl.pallas_call(
    _layer4_body,
    grid=(N // _BR,),
    in_specs=[
        pl.BlockSpec((NC, 4, _BR, 16), lambda i: (0, 0, i, 0)),
        pl.BlockSpec((_BR, 16), lambda i: (i, 0)),
        pl.BlockSpec((1, HID), lambda i: (0, 0)),
        pl.BlockSpec((1, HID), lambda i: (0, 0)),
    ],
    out_specs=pl.BlockSpec((_BR, 16), lambda i: (i, 0)),
    out_shape=jax.ShapeDtypeStruct((N, 16), _f32),
)


def _final_body(r_ref, dinv_ref, b_ref, o_ref):
    i = pl.program_id(0)
    a = dinv_ref[:, 0:1] * (r_ref[0, 0, :, 0:1] + r_ref[1, 0, :, 0:1])
    v = jax.nn.sigmoid(a + b_ref[0, 0])

    @pl.when(i == 0)
    def _():
        o_ref[...] = jnp.zeros_like(o_ref)

    o_ref[...] += (jnp.sum(v) / N)[None, None]


_tc_final = pl.pallas_call(
    _final_body,
    grid=(N // _BR,),
    in_specs=[
        pl.BlockSpec((NC, 1, _BR, 16), lambda i: (0, 0, i, 0)),
        pl.BlockSpec((_BR, 16), lambda i: (i, 0)),
        pl.BlockSpec((1, 1), lambda i: (0, 0)),
    ],
    out_specs=pl.BlockSpec((1, 1), lambda i: (0, 0)),
    out_shape=jax.ShapeDtypeStruct((1, 1), _f32),
)


def kernel(vertex_features, edges, weights, W1, b1, W2, b2, W3, b3, W4, b4,
           W5, b5):
    src = edges[0].astype(jnp.int32)
    dst = edges[1].astype(jnp.int32)

    ar = jnp.arange(N, dtype=jnp.int32)
    zi = jnp.zeros((EPAD,), jnp.int32)
    srcP = jnp.concatenate([src, ar, zi])
    dstP = jnp.concatenate([dst, ar, zi])
    nrmD = jnp.concatenate([weights, jnp.zeros((N + EPAD,), _f32)])
    nrmP = jnp.concatenate(
        [weights, jnp.ones((N,), _f32), jnp.zeros((EPAD,), _f32)])

    ones_tbl = jnp.ones((N, 16), _f32)
    x0 = jnp.pad(vertex_features, ((0, 0), (0, 16 - 6)))
    W1p = jnp.pad(W1, ((0, 0), (0, 16 - 6)))

    degm = _sc_prop1(srcP, dstP, nrmD, ones_tbl)
    dinv, g1 = _tc_dinv(degm, x0)
    r1 = _sc_prop1(srcP, dstP, nrmP, g1)
    g2 = _tc_layer1(r1, dinv, W1p, b1.reshape(1, HID), W2)
    r2 = _sc_prop4(srcP, dstP, nrmP, *g2)
    g3 = _tc_layer_mid(r2, dinv, b2.reshape(1, HID), W3)
    r3 = _sc_prop4(srcP, dstP, nrmP, *g3)
    g4 = _tc_layer_mid(r3, dinv, b3.reshape(1, HID), W4)
    r4 = _sc_prop4(srcP, dstP, nrmP, *g4)
    g5 = _tc_layer4(r4, dinv, b4.reshape(1, HID), W5)
    r5 = _sc_prop1(srcP, dstP, nrmP, g5)
    return _tc_final(r5, dinv, b5.reshape(1, 1))


# speedup vs baseline: 12.4348x; 1.1389x over previous
import functools

import jax
import jax.numpy as jnp
from jax import lax
from jax.experimental import pallas as pl
from jax.experimental.pallas import tpu as pltpu
from jax.experimental.pallas import tpu_sc as plsc

N = 100000
E = 1600000
HID = 64
RP = 13312
EPAD = RP * 128 - (E + N)
NC, NS = 2, 16
ROWS_SC = RP // NC
ROWS_TILE = ROWS_SC // NS
N2 = 100096
NTILE = N2 // NS
GROUP = 4
ZROWS = NTILE // 16

_MESH = plsc.VectorSubcoreMesh(
    core_axis_name="c", subcore_axis_name="s", num_cores=NC, num_subcores=NS)

_f32 = jnp.float32


def _prop_body(nchunks, *refs):
    (src_hbm, dst_hbm, nrm_hbm), tbls = refs[:3], refs[3:3 + nchunks]
    out_hbm = refs[3 + nchunks]
    rest = refs[4 + nchunks:]
    acc, zbuf = rest[0], rest[1]
    srcv = [rest[2 + s * GROUP:2 + (s + 1) * GROUP] for s in range(2)]
    o = 2 + 2 * GROUP
    dstv = [rest[o + s * GROUP:o + (s + 1) * GROUP] for s in range(2)]
    o += 2 * GROUP
    rows = [rest[o + s * GROUP:o + (s + 1) * GROUP] for s in range(2)]
    o += 2 * GROUP
    nrmv = rest[o:o + 2]
    semE = rest[o + 2:o + 4]
    semG = rest[o + 4:o + 6]
    cid = lax.axis_index("c")
    sid = lax.axis_index("s")

    def zb(i, _):
        zbuf[i, :] = jnp.zeros((16,), _f32)
        return 0

    lax.fori_loop(0, ZROWS, zb, 0)

    row0 = cid * ROWS_SC + sid * ROWS_TILE
    ngroups = ROWS_TILE // GROUP

    def fire_loads(tt, s):
        tt = jnp.minimum(tt, ngroups - 1)
        base = (row0 + tt * GROUP) * 128
        for g in range(GROUP):
            pltpu.async_copy(
                src_hbm.at[pl.ds(base + g * 128, 128)], srcv[s][g], semE[s])
            pltpu.async_copy(
                dst_hbm.at[pl.ds(base + g * 128, 128)], dstv[s][g], semE[s])
        pltpu.async_copy(
            nrm_hbm.at[pl.ds(base, GROUP * 128)], nrmv[s], semE[s])

    def drain_loads(s):
        for g in range(GROUP):
            pltpu.make_async_copy(
                src_hbm.at[pl.ds(0, 128)], srcv[s][g], semE[s]).wait()
            pltpu.make_async_copy(
                dst_hbm.at[pl.ds(0, 128)], dstv[s][g], semE[s]).wait()
        pltpu.make_async_copy(
            nrm_hbm.at[pl.ds(0, GROUP * 128)], nrmv[s], semE[s]).wait()

    for c in range(nchunks):
        tbl = tbls[c]
        for z in range(16):
            pltpu.sync_copy(zbuf, acc.at[pl.ds(sid * NTILE + z * ZROWS, ZROWS)])
        plsc.subcore_barrier()

        def fire_gathers(s):
            for g in range(GROUP):
                pltpu.async_copy(tbl.at[srcv[s][g]], rows[s][g], semG[s])

        def process(s):
            for g in range(GROUP):
                pltpu.make_async_copy(
                    tbl.at[srcv[s][g]], rows[s][g], semG[s]).wait()
                for k in range(8):
                    nv = nrmv[s][pl.ds((g * 8 + k) * 16, 16)]
                    for j in range(16):
                        rows[s][g][k * 16 + j, :] = (
                            rows[s][g][k * 16 + j, :] * nv[j])
                pltpu.sync_copy(rows[s][g], acc.at[dstv[s][g]], add=True)

        fire_loads(0, 0)

        def body(i, _):
            t0 = i * 2
            drain_loads(0)
            fire_gathers(0)
            fire_loads(t0 + 1, 1)
            process(0)
            drain_loads(1)
            fire_gathers(1)
            fire_loads(t0 + 2, 0)
            process(1)
            return 0

        lax.fori_loop(0, ngroups // 2, body, 0)
        drain_loads(0)
        plsc.subcore_barrier()
        pltpu.sync_copy(acc.at[pl.ds(sid * NTILE, NTILE)],
                        out_hbm.at[cid, c, pl.ds(sid * NTILE, NTILE)])


def _make_prop(nchunks):
    return pl.kernel(
        functools.partial(_prop_body, nchunks),
        out_type=jax.ShapeDtypeStruct((NC, nchunks, N2, 16), _f32),
        mesh=_MESH,
        compiler_params=pltpu.CompilerParams(use_tc_tiling_on_sc=False),
        scratch_types=(
            [pltpu.VMEM_SHARED((N2, 16), _f32),
             pltpu.VMEM((ZROWS, 16), _f32)]
            + [pltpu.VMEM((128,), jnp.int32) for _ in range(4 * GROUP)]
            + [pltpu.VMEM((128, 16), _f32) for _ in range(2 * GROUP)]
            + [pltpu.VMEM((GROUP * 128,), _f32) for _ in range(2)]
            + [pltpu.SemaphoreType.DMA for _ in range(4)]
        ),
    )


_sc_prop1 = _make_prop(1)
_sc_prop4 = _make_prop(4)


_BR = 1000


def _dinv_body(degm_ref, x0_ref, dinv_ref, g1_ref):
    deg = degm_ref[0, 0] + degm_ref[1, 0] + 1.0
    dinv = lax.rsqrt(deg)
    dinv_ref[...] = dinv
    g1_ref[...] = dinv * x0_ref[...]


_tc_dinv = pl.pallas_call(
    _dinv_body,
    grid=(N // _BR,),
    in_specs=[
        pl.BlockSpec((NC, 1, _BR, 16), lambda i: (0, 0, i, 0)),
        pl.BlockSpec((_BR, 16), lambda i: (i, 0)),
    ],
    out_specs=[
        pl.BlockSpec((_BR, 16), lambda i: (i, 0)),
        pl.BlockSpec((_BR, 16), lambda i: (i, 0)),
    ],
    out_shape=[
        jax.ShapeDtypeStruct((N, 16), _f32),
        jax.ShapeDtypeStruct((N, 16), _f32),
    ],
)


def _layer1_body(r_ref, dinv_ref, w1_ref, b1_ref, w2_ref, *o_refs):
    dinv = dinv_ref[...]
    t = dinv * (r_ref[0, 0] + r_ref[1, 0])
    x1 = jax.nn.sigmoid(
        jnp.dot(t, w1_ref[...].T, preferred_element_type=_f32) + b1_ref[...])
    h2 = jnp.dot(x1, w2_ref[...].T, preferred_element_type=_f32)
    for c in range(4):
        o_refs[c][...] = dinv * h2[:, c * 16:(c + 1) * 16]


_tc_layer1 = pl.pallas_call(
    _layer1_body,
    grid=(N // _BR,),
    in_specs=[
        pl.BlockSpec((NC, 1, _BR, 16), lambda i: (0, 0, i, 0)),
        pl.BlockSpec((_BR, 16), lambda i: (i, 0)),
        pl.BlockSpec((HID, 16), lambda i: (0, 0)),
        pl.BlockSpec((1, HID), lambda i: (0, 0)),
        pl.BlockSpec((HID, HID), lambda i: (0, 0)),
    ],
    out_specs=[pl.BlockSpec((_BR, 16), lambda i: (i, 0)) for _ in range(4)],
    out_shape=[jax.ShapeDtypeStruct((N, 16), _f32) for _ in range(4)],
)


def _layer_mid_body(r_ref, dinv_ref, b_ref, w_ref, *o_refs):
    dinv = dinv_ref[...]
    t = jnp.concatenate(
        [dinv * (r_ref[0, c] + r_ref[1, c]) for c in range(4)], axis=-1)
    x = jax.nn.sigmoid(t + b_ref[...])
    h = jnp.dot(x, w_ref[...].T, preferred_element_type=_f32)
    for c in range(4):
        o_refs[c][...] = dinv * h[:, c * 16:(c + 1) * 16]


_tc_layer_mid = pl.pallas_call(
    _layer_mid_body,
    grid=(N // _BR,),
    in_specs=[
        pl.BlockSpec((NC, 4, _BR, 16), lambda i: (0, 0, i, 0)),
        pl.BlockSpec((_BR, 16), lambda i: (i, 0)),
        pl.BlockSpec((1, HID), lambda i: (0, 0)),
        pl.BlockSpec((HID, HID), lambda i: (0, 0)),
    ],
    out_specs=[pl.BlockSpec((_BR, 16), lambda i: (i, 0)) for _ in range(4)],
    out_shape=[jax.ShapeDtypeStruct((N, 16), _f32) for _ in range(4)],
)


def _layer4_body(r_ref, dinv_ref, b_ref, w5_ref, o_ref):
    dinv = dinv_ref[...]
    t = jnp.concatenate(
        [dinv * (r_ref[0, c] + r_ref[1, c]) for c in range(4)], axis=-1)
    x = jax.nn.sigmoid(t + b_ref[...])
    h5 = jnp.dot(x, w5_ref[...].T, preferred_element_type=_f32)
    o_ref[...] = jnp.concatenate(
        [dinv[:, 0:1] * h5, jnp.zeros((_BR, 15), _f32)], axis=-1)


_tc_layer4 = pl.pallas_call(
    _layer4_body,
    grid=(N // _BR,),
    in_specs=[
        pl.BlockSpec((NC, 4, _BR, 16), lambda i: (0, 0, i, 0)),
        pl.BlockSpec((_BR, 16), lambda i: (i, 0)),
        pl.BlockSpec((1, HID), lambda i: (0, 0)),
        pl.BlockSpec((1, HID), lambda i: (0, 0)),
    ],
    out_specs=pl.BlockSpec((_BR, 16), lambda i: (i, 0)),
    out_shape=jax.ShapeDtypeStruct((N, 16), _f32),
)


def _final_body(r_ref, dinv_ref, b_ref, o_ref):
    i = pl.program_id(0)
    a = dinv_ref[:, 0:1] * (r_ref[0, 0, :, 0:1] + r_ref[1, 0, :, 0:1])
    v = jax.nn.sigmoid(a + b_ref[0, 0])

    @pl.when(i == 0)
    def _():
        o_ref[...] = jnp.zeros_like(o_ref)

    o_ref[...] += (jnp.sum(v) / N)[None, None]


_tc_final = pl.pallas_call(
    _final_body,
    grid=(N // _BR,),
    in_specs=[
        pl.BlockSpec((NC, 1, _BR, 16), lambda i: (0, 0, i, 0)),
        pl.BlockSpec((_BR, 16), lambda i: (i, 0)),
        pl.BlockSpec((1, 1), lambda i: (0, 0)),
    ],
    out_specs=pl.BlockSpec((1, 1), lambda i: (0, 0)),
    out_shape=jax.ShapeDtypeStruct((1, 1), _f32),
)


def kernel(vertex_features, edges, weights, W1, b1, W2, b2, W3, b3, W4, b4,
           W5, b5):
    src = edges[0].astype(jnp.int32)
    dst = edges[1].astype(jnp.int32)

    ar = jnp.arange(N, dtype=jnp.int32)
    zi = jnp.zeros((EPAD,), jnp.int32)
    srcP = jnp.concatenate([src, ar, zi])
    dstP = jnp.concatenate([dst, ar, zi])
    nrmD = jnp.concatenate([weights, jnp.zeros((N + EPAD,), _f32)])
    nrmP = jnp.concatenate(
        [weights, jnp.ones((N,), _f32), jnp.zeros((EPAD,), _f32)])

    ones_tbl = jnp.ones((N, 16), _f32)
    x0 = jnp.pad(vertex_features, ((0, 0), (0, 16 - 6)))
    W1p = jnp.pad(W1, ((0, 0), (0, 16 - 6)))

    degm = _sc_prop1(srcP, dstP, nrmD, ones_tbl)
    dinv, g1 = _tc_dinv(degm, x0)
    r1 = _sc_prop1(srcP, dstP, nrmP, g1)
    g2 = _tc_layer1(r1, dinv, W1p, b1.reshape(1, HID), W2)
    r2 = _sc_prop4(srcP, dstP, nrmP, *g2)
    g3 = _tc_layer_mid(r2, dinv, b2.reshape(1, HID), W3)
    r3 = _sc_prop4(srcP, dstP, nrmP, *g3)
    g4 = _tc_layer_mid(r3, dinv, b3.reshape(1, HID), W4)
    r4 = _sc_prop4(srcP, dstP, nrmP, *g4)
    g5 = _tc_layer4(r4, dinv, b4.reshape(1, HID), W5)
    r5 = _sc_prop1(srcP, dstP, nrmP, g5)
    return _tc_final(r5, dinv, b5.reshape(1, 1))


# speedup vs baseline: 12.7199x; 1.0229x over previous
import functools

import jax
import jax.numpy as jnp
from jax import lax
from jax.experimental import pallas as pl
from jax.experimental.pallas import tpu as pltpu
from jax.experimental.pallas import tpu_sc as plsc

N = 100000
E = 1600000
HID = 64
RP = 13312
EPAD = RP * 128 - (E + N)
NC, NS = 2, 16
ROWS_SC = RP // NC
ROWS_TILE = ROWS_SC // NS
N2 = 100096
NTILE = N2 // NS
GROUP = 4
ZROWS = NTILE // 16

_MESH = plsc.VectorSubcoreMesh(
    core_axis_name="c", subcore_axis_name="s", num_cores=NC, num_subcores=NS)

_f32 = jnp.float32


def _prop_body(nchunks, *refs):
    (src_hbm, dst_hbm, nrm_hbm), tbls = refs[:3], refs[3:3 + nchunks]
    out_hbm = refs[3 + nchunks]
    rest = refs[4 + nchunks:]
    acc, zbuf = rest[0], rest[1]
    srcv = [rest[2 + s * GROUP:2 + (s + 1) * GROUP] for s in range(2)]
    o = 2 + 2 * GROUP
    dstv = [rest[o + s * GROUP:o + (s + 1) * GROUP] for s in range(2)]
    o += 2 * GROUP
    rows = [rest[o + s * GROUP:o + (s + 1) * GROUP] for s in range(2)]
    o += 2 * GROUP
    nrmv = rest[o:o + 2]
    semE = rest[o + 2:o + 4]
    semG = rest[o + 4:o + 6]
    cid = lax.axis_index("c")
    sid = lax.axis_index("s")

    def zb(i, _):
        zbuf[i, :] = jnp.zeros((16,), _f32)
        return 0

    lax.fori_loop(0, ZROWS, zb, 0)

    row0 = cid * ROWS_SC + sid * ROWS_TILE
    ngroups = ROWS_TILE // GROUP

    def fire_loads(tt, s):
        tt = jnp.minimum(tt, ngroups - 1)
        base = (row0 + tt * GROUP) * 128
        for g in range(GROUP):
            pltpu.async_copy(
                src_hbm.at[pl.ds(base + g * 128, 128)], srcv[s][g], semE[s])
            pltpu.async_copy(
                dst_hbm.at[pl.ds(base + g * 128, 128)], dstv[s][g], semE[s])
        pltpu.async_copy(
            nrm_hbm.at[pl.ds(base, GROUP * 128)], nrmv[s], semE[s])

    def drain_loads(s):
        for g in range(GROUP):
            pltpu.make_async_copy(
                src_hbm.at[pl.ds(0, 128)], srcv[s][g], semE[s]).wait()
            pltpu.make_async_copy(
                dst_hbm.at[pl.ds(0, 128)], dstv[s][g], semE[s]).wait()
        pltpu.make_async_copy(
            nrm_hbm.at[pl.ds(0, GROUP * 128)], nrmv[s], semE[s]).wait()

    for c in range(nchunks):
        tbl = tbls[c]
        for z in range(16):
            pltpu.sync_copy(zbuf, acc.at[pl.ds(sid * NTILE + z * ZROWS, ZROWS)])
        plsc.subcore_barrier()

        def fire_gathers(s):
            for g in range(GROUP):
                pltpu.async_copy(tbl.at[srcv[s][g]], rows[s][g], semG[s])

        def process(s):
            for g in range(GROUP):
                pltpu.make_async_copy(
                    tbl.at[srcv[s][g]], rows[s][g], semG[s]).wait()
                for k in range(8):
                    nv = nrmv[s][pl.ds((g * 8 + k) * 16, 16)]
                    for j in range(16):
                        rows[s][g][k * 16 + j, :] = (
                            rows[s][g][k * 16 + j, :] * nv[j])
                pltpu.sync_copy(rows[s][g], acc.at[dstv[s][g]], add=True)

        fire_loads(0, 0)

        def body(i, _):
            t0 = i * 2
            drain_loads(0)
            fire_gathers(0)
            fire_loads(t0 + 1, 1)
            process(0)
            drain_loads(1)
            fire_gathers(1)
            fire_loads(t0 + 2, 0)
            process(1)
            return 0

        lax.fori_loop(0, ngroups // 2, body, 0)
        drain_loads(0)
        plsc.subcore_barrier()
        pltpu.sync_copy(acc.at[pl.ds(sid * NTILE, NTILE)],
                        out_hbm.at[cid, c, pl.ds(sid * NTILE, NTILE)])


def _make_prop(nchunks):
    return pl.kernel(
        functools.partial(_prop_body, nchunks),
        out_type=jax.ShapeDtypeStruct((NC, nchunks, N2, 16), _f32),
        mesh=_MESH,
        compiler_params=pltpu.CompilerParams(use_tc_tiling_on_sc=False),
        scratch_types=(
            [pltpu.VMEM_SHARED((N2, 16), _f32),
             pltpu.VMEM((ZROWS, 16), _f32)]
            + [pltpu.VMEM((128,), jnp.int32) for _ in range(4 * GROUP)]
            + [pltpu.VMEM((128, 16), _f32) for _ in range(2 * GROUP)]
            + [pltpu.VMEM((GROUP * 128,), _f32) for _ in range(2)]
            + [pltpu.SemaphoreType.DMA for _ in range(4)]
        ),
    )


_sc_prop1 = _make_prop(1)
_sc_prop4 = _make_prop(4)


def _deg_body(src_hbm, dst_hbm, nrm_hbm, out_hbm, *rest):
    acc, zbuf = rest[0], rest[1]
    dstv = [rest[2 + s * GROUP:2 + (s + 1) * GROUP] for s in range(2)]
    o = 2 + 2 * GROUP
    rows = [rest[o + s * GROUP:o + (s + 1) * GROUP] for s in range(2)]
    o += 2 * GROUP
    nrmv = rest[o:o + 2]
    semE = rest[o + 2:o + 4]
    cid = lax.axis_index("c")
    sid = lax.axis_index("s")

    def zb(i, _):
        zbuf[i, :] = jnp.zeros((16,), _f32)
        return 0

    lax.fori_loop(0, ZROWS, zb, 0)
    for z in range(16):
        pltpu.sync_copy(zbuf, acc.at[pl.ds(sid * NTILE + z * ZROWS, ZROWS)])
    plsc.subcore_barrier()

    row0 = cid * ROWS_SC + sid * ROWS_TILE
    ngroups = ROWS_TILE // GROUP

    def fire_loads(tt, s):
        tt = jnp.minimum(tt, ngroups - 1)
        base = (row0 + tt * GROUP) * 128
        for g in range(GROUP):
            pltpu.async_copy(
                dst_hbm.at[pl.ds(base + g * 128, 128)], dstv[s][g], semE[s])
        pltpu.async_copy(
            nrm_hbm.at[pl.ds(base, GROUP * 128)], nrmv[s], semE[s])

    def drain_loads(s):
        for g in range(GROUP):
            pltpu.make_async_copy(
                dst_hbm.at[pl.ds(0, 128)], dstv[s][g], semE[s]).wait()
        pltpu.make_async_copy(
            nrm_hbm.at[pl.ds(0, GROUP * 128)], nrmv[s], semE[s]).wait()

    def process(s):
        for g in range(GROUP):
            for k in range(8):
                nv = nrmv[s][pl.ds((g * 8 + k) * 16, 16)]
                for j in range(16):
                    rows[s][g][k * 16 + j, :] = jnp.ones((16,), _f32) * nv[j]
            pltpu.sync_copy(rows[s][g], acc.at[dstv[s][g]], add=True)

    fire_loads(0, 0)

    def body(i, _):
        t0 = i * 2
        drain_loads(0)
        fire_loads(t0 + 1, 1)
        process(0)
        drain_loads(1)
        fire_loads(t0 + 2, 0)
        process(1)
        return 0

    lax.fori_loop(0, ngroups // 2, body, 0)
    drain_loads(0)
    plsc.subcore_barrier()
    pltpu.sync_copy(acc.at[pl.ds(sid * NTILE, NTILE)],
                    out_hbm.at[cid, 0, pl.ds(sid * NTILE, NTILE)])


_sc_deg = pl.kernel(
    _deg_body,
    out_type=jax.ShapeDtypeStruct((NC, 1, N2, 16), _f32),
    mesh=_MESH,
    compiler_params=pltpu.CompilerParams(use_tc_tiling_on_sc=False),
    scratch_types=(
        [pltpu.VMEM_SHARED((N2, 16), _f32),
         pltpu.VMEM((ZROWS, 16), _f32)]
        + [pltpu.VMEM((128,), jnp.int32) for _ in range(2 * GROUP)]
        + [pltpu.VMEM((128, 16), _f32) for _ in range(2 * GROUP)]
        + [pltpu.VMEM((GROUP * 128,), _f32) for _ in range(2)]
        + [pltpu.SemaphoreType.DMA for _ in range(2)]
    ),
)


_BR = 1000


def _dinv_body(degm_ref, x0_ref, dinv_ref, g1_ref):
    deg = degm_ref[0, 0] + degm_ref[1, 0] + 1.0
    dinv = lax.rsqrt(deg)
    dinv_ref[...] = dinv
    g1_ref[...] = dinv * x0_ref[...]


_tc_dinv = pl.pallas_call(
    _dinv_body,
    grid=(N // _BR,),
    in_specs=[
        pl.BlockSpec((NC, 1, _BR, 16), lambda i: (0, 0, i, 0)),
        pl.BlockSpec((_BR, 16), lambda i: (i, 0)),
    ],
    out_specs=[
        pl.BlockSpec((_BR, 16), lambda i: (i, 0)),
        pl.BlockSpec((_BR, 16), lambda i: (i, 0)),
    ],
    out_shape=[
        jax.ShapeDtypeStruct((N, 16), _f32),
        jax.ShapeDtypeStruct((N, 16), _f32),
    ],
)


def _layer1_body(r_ref, dinv_ref, w1_ref, b1_ref, w2_ref, *o_refs):
    dinv = dinv_ref[...]
    t = dinv * (r_ref[0, 0] + r_ref[1, 0])
    x1 = jax.nn.sigmoid(
        jnp.dot(t, w1_ref[...].T, preferred_element_type=_f32) + b1_ref[...])
    h2 = jnp.dot(x1, w2_ref[...].T, preferred_element_type=_f32)
    for c in range(4):
        o_refs[c][...] = dinv * h2[:, c * 16:(c + 1) * 16]


_tc_layer1 = pl.pallas_call(
    _layer1_body,
    grid=(N // _BR,),
    in_specs=[
        pl.BlockSpec((NC, 1, _BR, 16), lambda i: (0, 0, i, 0)),
        pl.BlockSpec((_BR, 16), lambda i: (i, 0)),
        pl.BlockSpec((HID, 16), lambda i: (0, 0)),
        pl.BlockSpec((1, HID), lambda i: (0, 0)),
        pl.BlockSpec((HID, HID), lambda i: (0, 0)),
    ],
    out_specs=[pl.BlockSpec((_BR, 16), lambda i: (i, 0)) for _ in range(4)],
    out_shape=[jax.ShapeDtypeStruct((N, 16), _f32) for _ in range(4)],
)


def _layer_mid_body(r_ref, dinv_ref, b_ref, w_ref, *o_refs):
    dinv = dinv_ref[...]
    t = jnp.concatenate(
        [dinv * (r_ref[0, c] + r_ref[1, c]) for c in range(4)], axis=-1)
    x = jax.nn.sigmoid(t + b_ref[...])
    h = jnp.dot(x, w_ref[...].T, preferred_element_type=_f32)
    for c in range(4):
        o_refs[c][...] = dinv * h[:, c * 16:(c + 1) * 16]


_tc_layer_mid = pl.pallas_call(
    _layer_mid_body,
    grid=(N // _BR,),
    in_specs=[
        pl.BlockSpec((NC, 4, _BR, 16), lambda i: (0, 0, i, 0)),
        pl.BlockSpec((_BR, 16), lambda i: (i, 0)),
        pl.BlockSpec((1, HID), lambda i: (0, 0)),
        pl.BlockSpec((HID, HID), lambda i: (0, 0)),
    ],
    out_specs=[pl.BlockSpec((_BR, 16), lambda i: (i, 0)) for _ in range(4)],
    out_shape=[jax.ShapeDtypeStruct((N, 16), _f32) for _ in range(4)],
)


def _layer4_body(r_ref, dinv_ref, b_ref, w5_ref, o_ref):
    dinv = dinv_ref[...]
    t = jnp.concatenate(
        [dinv * (r_ref[0, c] + r_ref[1, c]) for c in range(4)], axis=-1)
    x = jax.nn.sigmoid(t + b_ref[...])
    h5 = jnp.dot(x, w5_ref[...].T, preferred_element_type=_f32)
    o_ref[...] = jnp.concatenate(
        [dinv[:, 0:1] * h5, jnp.zeros((_BR, 15), _f32)], axis=-1)


_tc_layer4 = pl.pallas_call(
    _layer4_body,
    grid=(N // _BR,),
    in_specs=[
        pl.BlockSpec((NC, 4, _BR, 16), lambda i: (0, 0, i, 0)),
        pl.BlockSpec((_BR, 16), lambda i: (i, 0)),
        pl.BlockSpec((1, HID), lambda i: (0, 0)),
        pl.BlockSpec((1, HID), lambda i: (0, 0)),
    ],
    out_specs=pl.BlockSpec((_BR, 16), lambda i: (i, 0)),
    out_shape=jax.ShapeDtypeStruct((N, 16), _f32),
)


def _final_body(r_ref, dinv_ref, b_ref, o_ref):
    i = pl.program_id(0)
    a = dinv_ref[:, 0:1] * (r_ref[0, 0, :, 0:1] + r_ref[1, 0, :, 0:1])
    v = jax.nn.sigmoid(a + b_ref[0, 0])

    @pl.when(i == 0)
    def _():
        o_ref[...] = jnp.zeros_like(o_ref)

    o_ref[...] += (jnp.sum(v) / N)[None, None]


_tc_final = pl.pallas_call(
    _final_body,
    grid=(N // _BR,),
    in_specs=[
        pl.BlockSpec((NC, 1, _BR, 16), lambda i: (0, 0, i, 0)),
        pl.BlockSpec((_BR, 16), lambda i: (i, 0)),
        pl.BlockSpec((1, 1), lambda i: (0, 0)),
    ],
    out_specs=pl.BlockSpec((1, 1), lambda i: (0, 0)),
    out_shape=jax.ShapeDtypeStruct((1, 1), _f32),
)


def kernel(vertex_features, edges, weights, W1, b1, W2, b2, W3, b3, W4, b4,
           W5, b5):
    src = edges[0].astype(jnp.int32)
    dst = edges[1].astype(jnp.int32)

    ar = jnp.arange(N, dtype=jnp.int32)
    zi = jnp.zeros((EPAD,), jnp.int32)
    srcP = jnp.concatenate([src, ar, zi])
    dstP = jnp.concatenate([dst, ar, zi])
    nrmD = jnp.concatenate([weights, jnp.zeros((N + EPAD,), _f32)])
    nrmP = jnp.concatenate(
        [weights, jnp.ones((N,), _f32), jnp.zeros((EPAD,), _f32)])

    x0 = jnp.pad(vertex_features, ((0, 0), (0, 16 - 6)))
    W1p = jnp.pad(W1, ((0, 0), (0, 16 - 6)))

    degm = _sc_deg(srcP, dstP, nrmD)
    dinv, g1 = _tc_dinv(degm, x0)
    r1 = _sc_prop1(srcP, dstP, nrmP, g1)
    g2 = _tc_layer1(r1, dinv, W1p, b1.reshape(1, HID), W2)
    r2 = _sc_prop4(srcP, dstP, nrmP, *g2)
    g3 = _tc_layer_mid(r2, dinv, b2.reshape(1, HID), W3)
    r3 = _sc_prop4(srcP, dstP, nrmP, *g3)
    g4 = _tc_layer_mid(r3, dinv, b3.reshape(1, HID), W4)
    r4 = _sc_prop4(srcP, dstP, nrmP, *g4)
    g5 = _tc_layer4(r4, dinv, b4.reshape(1, HID), W5)
    r5 = _sc_prop1(srcP, dstP, nrmP, g5)
    return _tc_final(r5, dinv, b5.reshape(1, 1))
